# Initial kernel scaffold; baseline (speedup 1.0000x reference)
import jax
import jax.numpy as jnp
from jax.experimental import pallas as pl


def kernel(features, edge_index, W0, b0, W1, b1, W2, b2):
    raise NotImplementedError("write your pallas kernel here")


# speedup vs baseline: 3.3897x; 3.3897x over previous
import functools

import jax
import jax.numpy as jnp
from jax import lax
from jax.experimental import pallas as pl
from jax.experimental.pallas import tpu as pltpu
from jax.experimental.pallas import tpu_sc as plsc

N_NODES = 10000
N_EDGES = 160000
NUM_TILES = 16
NUM_CORES = 2
EDGE_BLOCK = 128
EDGES_PER_TILE = 10240
N_HALVES = 2
NB_H = EDGES_PER_TILE // (N_HALVES * EDGE_BLOCK)
E_PAD = NUM_TILES * EDGES_PER_TILE
ACC_ROWS = 10112
INIT_ROWS = ACC_ROWS // NUM_TILES
OUT_ROWS = 624
DUMMY_ROW = N_NODES


def _sc_agg_body(h_ref, srcp_ref, dst_ref, init_ref, out_ref,
                 acc, src_v, dst_v, rows_a, rows_b, sem_a, sem_b):
    cid = lax.axis_index("c")
    tid = lax.axis_index("s")
    pltpu.sync_copy(init_ref, acc.at[pl.ds(tid * INIT_ROWS, INIT_ROWS)])
    plsc.subcore_barrier()

    for half in range(N_HALVES):
        pltpu.sync_copy(srcp_ref.at[cid, tid, half], src_v)
        pltpu.sync_copy(dst_ref.at[tid, half], dst_v)
        pltpu.async_copy(h_ref.at[src_v.at[0]], rows_a, sem_a)

        def body(i, carry):
            j = 2 * i
            pltpu.async_copy(h_ref.at[src_v.at[j + 1]], rows_b, sem_b)
            pltpu.make_async_copy(h_ref.at[src_v.at[j]], rows_a, sem_a).wait()
            pltpu.sync_copy(rows_a, acc.at[dst_v.at[j]], add=True)
            pltpu.async_copy(h_ref.at[src_v.at[j + 2]], rows_a, sem_a)
            pltpu.make_async_copy(
                h_ref.at[src_v.at[j + 1]], rows_b, sem_b).wait()
            pltpu.sync_copy(rows_b, acc.at[dst_v.at[j + 1]], add=True)
            return carry
        lax.fori_loop(0, (NB_H - 2) // 2, body, 0)
        j = NB_H - 2
        pltpu.async_copy(h_ref.at[src_v.at[j + 1]], rows_b, sem_b)
        pltpu.make_async_copy(h_ref.at[src_v.at[j]], rows_a, sem_a).wait()
        pltpu.sync_copy(rows_a, acc.at[dst_v.at[j]], add=True)
        pltpu.make_async_copy(h_ref.at[src_v.at[j + 1]], rows_b, sem_b).wait()
        pltpu.sync_copy(rows_b, acc.at[dst_v.at[j + 1]], add=True)

    plsc.subcore_barrier()
    @pl.when(tid < NUM_TILES - 1)
    def _():
        pltpu.sync_copy(
            acc.at[pl.ds(tid * OUT_ROWS, OUT_ROWS)],
            out_ref.at[pl.ds(cid * N_NODES + tid * OUT_ROWS, OUT_ROWS)])

    @pl.when(tid == NUM_TILES - 1)
    def _():
        last = (NUM_TILES - 1) * OUT_ROWS
        pltpu.sync_copy(
            acc.at[pl.ds(last, N_NODES - last)],
            out_ref.at[pl.ds(cid * N_NODES + last, N_NODES - last)])


@functools.partial(
    pl.kernel,
    out_type=jax.ShapeDtypeStruct((NUM_CORES * N_NODES, 128), jnp.float32),
    mesh=plsc.VectorSubcoreMesh(core_axis_name="c", subcore_axis_name="s"),
    scratch_types=[
        pltpu.VMEM_SHARED((ACC_ROWS, 128), jnp.float32),
        pltpu.VMEM((NB_H, EDGE_BLOCK), jnp.int32),
        pltpu.VMEM((NB_H, EDGE_BLOCK), jnp.int32),
        pltpu.VMEM((EDGE_BLOCK, 128), jnp.float32),
        pltpu.VMEM((EDGE_BLOCK, 128), jnp.float32),
        pltpu.SemaphoreType.DMA,
        pltpu.SemaphoreType.DMA,
    ],
)
def _sc_aggregate(h_ref, srcp_ref, dst_ref, init_ref, out_ref,
                  acc, src_v, dst_v, rows_a, rows_b, sem_a, sem_b):
    _sc_agg_body(h_ref, srcp_ref, dst_ref, init_ref, out_ref,
                 acc, src_v, dst_v, rows_a, rows_b, sem_a, sem_b)


def _mm_body(oc, bn, relu, a_ref, w_ref, b_ref, o_ref):
    c = pl.program_id(1)
    nc = pl.num_programs(1)

    @pl.when(c == 0)
    def _():
        for o in range(oc):
            o_ref[o] = jnp.broadcast_to(b_ref[o], (bn, 128))

    m = jnp.dot(a_ref[...], w_ref[...], preferred_element_type=jnp.float32)
    for o in range(oc):
        o_ref[o] += m[:, o * 128:(o + 1) * 128]

    if relu:
        @pl.when(c == nc - 1)
        def _():
            for o in range(oc):
                o_ref[o] = jnp.maximum(o_ref[o], 0.0)


def _tc_matmul(a3, w3, bias, relu):
    cc, n, _ = a3.shape
    o_full = w3.shape[2]
    oc = o_full // 128
    bn = 1000
    grid = (n // bn, cc)
    bias3 = bias.reshape(oc, 1, 128)
    return pl.pallas_call(
        functools.partial(_mm_body, oc, bn, relu),
        grid=grid,
        in_specs=[
            pl.BlockSpec((None, bn, 128), lambda nb, c: (c, nb, 0)),
            pl.BlockSpec((None, 128, o_full), lambda nb, c: (c, 0, 0)),
            pl.BlockSpec((oc, 1, 128), lambda nb, c: (0, 0, 0)),
        ],
        out_specs=pl.BlockSpec((oc, bn, 128), lambda nb, c: (0, nb, 0)),
        out_shape=jax.ShapeDtypeStruct((oc, n, 128), jnp.float32),
    )(a3, w3, bias3)


def kernel(features, edge_index, W0, b0, W1, b1, W2, b2):
    n, f_in = features.shape
    src = edge_index[0]
    dst = edge_index[1]
    pad = E_PAD - N_EDGES
    src_p = jnp.concatenate([src, jnp.zeros((pad,), jnp.int32)])
    dst_p = jnp.concatenate([dst, jnp.full((pad,), DUMMY_ROW, jnp.int32)])
    srcp = jnp.stack([src_p, src_p + n]).reshape(
        NUM_CORES, NUM_TILES, N_HALVES, NB_H, EDGE_BLOCK)
    dst3 = dst_p.reshape(NUM_TILES, N_HALVES, NB_H, EDGE_BLOCK)
    zero_init = jnp.zeros((INIT_ROWS, 128), jnp.float32)
    b2_init = jnp.broadcast_to(b2, (INIT_ROWS, 128))

    x3 = features.reshape(n, 2, 128).transpose(1, 0, 2)
    a0 = _sc_aggregate(x3.reshape(2 * n, 128), srcp, dst3, zero_init)
    a0 = a0.reshape(2, n, 128)
    h1 = _tc_matmul(a0, W0.reshape(2, 128, -1), b0, relu=True)

    a1 = []
    for c in range(4):
        part = _sc_aggregate(
            h1[2 * c:2 * c + 2].reshape(2 * n, 128), srcp, dst3, zero_init)
        a1.append(part.reshape(2, n, 128))
    a1 = jnp.concatenate(a1, axis=0)
    h2 = _tc_matmul(a1, W1.reshape(8, 128, -1), b1, relu=True)

    t = _tc_matmul(h2, W2.reshape(8, 128, -1), jnp.zeros((128,), jnp.float32),
                   relu=False)
    tcat = jnp.concatenate([t[0], t[0]], axis=0)
    out = _sc_aggregate(tcat, srcp, dst3, b2_init)
    return out[:n]
